# baseline (speedup 1.0000x reference)
import functools

import jax
import jax.numpy as jnp
from jax import lax
from jax.experimental import pallas as pl
from jax.experimental.pallas import tpu as pltpu
from jax.experimental.pallas import tpu_sc as plsc

NC = 2
NS = 16
NW = NC * NS
L = 16

BATCH = 16384
DIM = 64
B_PER_W = BATCH // NW
CHUNK = 128
NCHUNK = B_PER_W // CHUNK

_MESH = dict(core_axis_name="c", subcore_axis_name="s",
             num_cores=NC, num_subcores=NS)
_CPARAMS = dict(needs_layout_passes=False, use_tc_tiling_on_sc=False)


def _gather_body(idx_hbm, w_hbm, out_hbm, idx_v, rows_v, sem):
    wid = lax.axis_index("s") * NC + lax.axis_index("c")
    base = wid * B_PER_W
    pltpu.sync_copy(idx_hbm.at[pl.ds(wid * NCHUNK, NCHUNK)], idx_v)
    copies = []
    for j in range(NCHUNK):
        copies.append(pltpu.async_copy(
            w_hbm.at[idx_v.at[j]], rows_v.at[pl.ds(j * CHUNK, CHUNK)], sem))
    for c in copies:
        c.wait()
    pltpu.sync_copy(rows_v, out_hbm.at[pl.ds(base, B_PER_W)])


def _dot_body(ce_hbm, xe_hbm, out_hbm, crows_v, xrows_v, out_v, sem):
    wid = lax.axis_index("s") * NC + lax.axis_index("c")
    base = wid * B_PER_W
    pltpu.sync_copy(ce_hbm.at[pl.ds(base, B_PER_W)], crows_v)
    pltpu.sync_copy(xe_hbm.at[pl.ds(base, B_PER_W)], xrows_v)
    iota = lax.iota(jnp.int32, L)

    def group(g, carry):
        r0 = g * L
        vec = jnp.zeros((L,), jnp.float32)
        for u in range(L):
            r = r0 + u
            s = jnp.zeros((L,), jnp.float32)
            for k in range(DIM // L):
                cg = crows_v[r, pl.ds(k * L, L)]
                xg = xrows_v[r, pl.ds(k * L, L)]
                s = s + cg * xg
            vec = jnp.where(iota == u, jnp.sum(s), vec)
        out_v[pl.ds(r0, L)] = vec
        return carry

    lax.fori_loop(0, B_PER_W // L, group, 0)
    pltpu.sync_copy(out_v, out_hbm.at[pl.ds(base, B_PER_W)])


@jax.jit
def _run(cidx, xidx, cw, xw):
    mesh = plsc.VectorSubcoreMesh(**_MESH)
    gather = pl.kernel(
        _gather_body,
        out_type=jax.ShapeDtypeStruct((BATCH, DIM), jnp.float32),
        mesh=mesh,
        compiler_params=pltpu.CompilerParams(**_CPARAMS),
        scratch_types=[
            pltpu.VMEM((NCHUNK, CHUNK), jnp.int32),
            pltpu.VMEM((B_PER_W, DIM), jnp.float32),
            pltpu.SemaphoreType.DMA,
        ],
    )
    ce = gather(cidx, cw)
    xe = gather(xidx, xw)
    dot = pl.kernel(
        _dot_body,
        out_type=jax.ShapeDtypeStruct((BATCH,), jnp.float32),
        mesh=mesh,
        compiler_params=pltpu.CompilerParams(**_CPARAMS),
        scratch_types=[
            pltpu.VMEM((B_PER_W, DIM), jnp.float32),
            pltpu.VMEM((B_PER_W, DIM), jnp.float32),
            pltpu.VMEM((B_PER_W,), jnp.float32),
            pltpu.SemaphoreType.DMA,
        ],
    )
    return dot(ce, xe)


def kernel(center_idx, context_idx, center_W, context_W):
    cidx = center_idx.astype(jnp.int32).reshape(NW * NCHUNK, CHUNK)
    xidx = context_idx.astype(jnp.int32).reshape(NW * NCHUNK, CHUNK)
    return _run(cidx, xidx, center_W, context_W)
